# Initial kernel scaffold; baseline (speedup 1.0000x reference)
import functools

import jax
import jax.numpy as jnp
from jax import lax
from jax.experimental import pallas as pl
from jax.experimental.pallas import tpu as pltpu
from jax.experimental.pallas import tpu_sc as plsc

N = 10000
E = 320000
D = 128
BETA_ = 0.5
C_ = 1.0

NC = 2
NS = 16
SUB = 125
GRP = 4
CHUNK = SUB * GRP
ROWS_PER_TILE = N // NS
CHUNK_ROWS = E // SUB
CR_PER_TILE = CHUNK_ROWS // (NC * NS)
OUTER = CR_PER_TILE // GRP


def _mm_body(x_ref, w_ref, o_ref):
    o_ref[...] = jnp.dot(x_ref[...], w_ref[...],
                         preferred_element_type=jnp.float32)


def _matmul(x, W):
    return pl.pallas_call(
        _mm_body,
        grid=(10,),
        in_specs=[
            pl.BlockSpec((N // 10, D), lambda i: (i, 0)),
            pl.BlockSpec((D, D), lambda i: (0, 0)),
        ],
        out_specs=pl.BlockSpec((N // 10, D), lambda i: (i, 0)),
        out_shape=jax.ShapeDtypeStruct((N, D), jnp.float32),
    )(x, W)


def _mix_body(p_ref, b_ref, o_ref):
    z = p_ref[0] + p_ref[1] + b_ref[...]
    o_ref[...] = BETA_ * z + (C_ - BETA_) * jnp.maximum(z, 0.0)


def _mix(partials, b):
    return pl.pallas_call(
        _mix_body,
        grid=(10,),
        in_specs=[
            pl.BlockSpec((2, N // 10, D), lambda i: (0, i, 0)),
            pl.BlockSpec((1, D), lambda i: (0, 0)),
        ],
        out_specs=pl.BlockSpec((N // 10, D), lambda i: (i, 0)),
        out_shape=jax.ShapeDtypeStruct((N, D), jnp.float32),
    )(partials, b.reshape(1, D))


def _sc_body(xlin, srcs, dsts, ews, out, acc, src_v, dst_v, ew_v, rows_v, sem):
    c = lax.axis_index("c")
    s = lax.axis_index("s")

    def zrow(i, carry):
        for cb in range(8):
            rows_v[i, pl.ds(cb * 16, 16)] = jnp.zeros((16,), jnp.float32)
        return carry
    lax.fori_loop(0, CHUNK, zrow, 0)
    base_n = s * ROWS_PER_TILE
    pltpu.sync_copy(rows_v, acc.at[pl.ds(base_n, CHUNK)])
    pltpu.sync_copy(rows_v.at[pl.ds(0, ROWS_PER_TILE - CHUNK)],
                    acc.at[pl.ds(base_n + CHUNK, ROWS_PER_TILE - CHUNK)])
    plsc.subcore_barrier()

    tile_cr0 = (c * NS + s) * CR_PER_TILE

    def outer(g, carry):
        base_cr = tile_cr0 + g * GRP
        pltpu.sync_copy(srcs.at[pl.ds(base_cr, GRP)], src_v)
        pltpu.sync_copy(dsts.at[pl.ds(base_cr, GRP)], dst_v)
        pltpu.sync_copy(ews.at[pl.ds(base_cr, GRP)], ew_v)
        descs = [
            pltpu.async_copy(xlin.at[src_v.at[j]],
                             rows_v.at[pl.ds(j * SUB, SUB)], sem)
            for j in range(GRP)
        ]
        for d in descs:
            d.wait()
        for j in range(GRP):
            def scale(e, carry2):
                w = ew_v[j, e]
                for cb in range(8):
                    r = rows_v[j * SUB + e, pl.ds(cb * 16, 16)]
                    rows_v[j * SUB + e, pl.ds(cb * 16, 16)] = r * w
                return carry2
            lax.fori_loop(0, SUB, scale, 0)
        for j in range(GRP):
            pltpu.sync_copy(rows_v.at[pl.ds(j * SUB, SUB)],
                            acc.at[dst_v.at[j]], add=True)
        return carry

    lax.fori_loop(0, OUTER, outer, 0)
    plsc.subcore_barrier()

    pltpu.sync_copy(acc.at[pl.ds(base_n, ROWS_PER_TILE)],
                    out.at[c, pl.ds(base_n, ROWS_PER_TILE)])


def _scatter_gather(xlin, srcs, dsts, ews):
    mesh = plsc.VectorSubcoreMesh(core_axis_name="c", subcore_axis_name="s")
    return pl.kernel(
        _sc_body,
        out_type=jax.ShapeDtypeStruct((NC, N, D), jnp.float32),
        mesh=mesh,
        scratch_types=[
            pltpu.VMEM_SHARED((N, D), jnp.float32),
            pltpu.VMEM((GRP, SUB), jnp.int32),
            pltpu.VMEM((GRP, SUB), jnp.int32),
            pltpu.VMEM((GRP, SUB), jnp.float32),
            pltpu.VMEM((CHUNK, D), jnp.float32),
            pltpu.SemaphoreType.DMA,
        ],
    )(xlin, srcs, dsts, ews)


def kernel(x, edge_index, edge_weight, W, b):
    src = edge_index[0].astype(jnp.int32).reshape(CHUNK_ROWS, SUB)
    dst = edge_index[1].astype(jnp.int32).reshape(CHUNK_ROWS, SUB)
    ew = edge_weight.reshape(CHUNK_ROWS, SUB)
    x_lin = _matmul(x, W)
    partials = _scatter_gather(x_lin, src, dst, ew)
    return _mix(partials, b)


# speedup vs baseline: 7.3676x; 7.3676x over previous
import jax
import jax.numpy as jnp
from jax import lax
from jax.experimental import pallas as pl
from jax.experimental.pallas import tpu as pltpu
from jax.experimental.pallas import tpu_sc as plsc

N = 10000
E = 320000
D = 128
BETA_ = 0.5
C_ = 1.0

NC = 2
NS = 16
NW = NC * NS
SUB = 128
CR = E // SUB
SLAB = 80
CR_PAD = SLAB * NW
N_PAD = 10240
ROWS_PER_TILE = N_PAD // NS


def _mm_body(x_ref, w_ref, o_ref):
    o_ref[...] = jnp.dot(x_ref[...], w_ref[...],
                         preferred_element_type=jnp.float32)


def _matmul(x, W):
    return pl.pallas_call(
        _mm_body,
        grid=(10,),
        in_specs=[
            pl.BlockSpec((N // 10, D), lambda i: (i, 0)),
            pl.BlockSpec((D, D), lambda i: (0, 0)),
        ],
        out_specs=pl.BlockSpec((N // 10, D), lambda i: (i, 0)),
        out_shape=jax.ShapeDtypeStruct((N, D), jnp.float32),
    )(x, W)


def _mix_body(p_ref, b_ref, o_ref):
    z = p_ref[0] + p_ref[1] + b_ref[...]
    o_ref[...] = BETA_ * z + (C_ - BETA_) * jnp.maximum(z, 0.0)


def _mix(partials, b):
    return pl.pallas_call(
        _mix_body,
        grid=(10,),
        in_specs=[
            pl.BlockSpec((2, N // 10, D), lambda i: (0, i, 0)),
            pl.BlockSpec((1, D), lambda i: (0, 0)),
        ],
        out_specs=pl.BlockSpec((N // 10, D), lambda i: (i, 0)),
        out_shape=jax.ShapeDtypeStruct((N, D), jnp.float32),
    )(partials, b.reshape(1, D))


def _sc_body(xlin, srcs, dsts, ews, out, acc, src_v, dst_v, ew_v, rows_v, sem):
    c = lax.axis_index("c")
    s = lax.axis_index("s")
    wid = c * NS + s

    def zrow(i, carry):
        for cb in range(8):
            rows_v[i, pl.ds(cb * 16, 16)] = jnp.zeros((16,), jnp.float32)
        return carry
    lax.fori_loop(0, SUB, zrow, 0)
    base_n = s * ROWS_PER_TILE
    for k in range(ROWS_PER_TILE // SUB):
        pltpu.sync_copy(rows_v, acc.at[pl.ds(base_n + k * SUB, SUB)])
    plsc.subcore_barrier()

    start = wid * SLAB
    cnt = jnp.minimum(SLAB, jnp.maximum(0, CR - wid * SLAB))
    pltpu.sync_copy(srcs.at[pl.ds(start, SLAB)], src_v)
    pltpu.sync_copy(dsts.at[pl.ds(start, SLAB)], dst_v)
    pltpu.sync_copy(ews.at[pl.ds(start, SLAB)], ew_v)

    def chunk_body(j, carry):
        pltpu.async_copy(xlin.at[src_v.at[j]], rows_v, sem).wait()

        def scale16(q, carry2):
            ewv = ew_v[j, pl.ds(q * 16, 16)]
            for e in range(16):
                wv = jnp.broadcast_to(ewv[e], (16,))
                for cb in range(8):
                    r = rows_v[q * 16 + e, pl.ds(cb * 16, 16)]
                    rows_v[q * 16 + e, pl.ds(cb * 16, 16)] = r * wv
            return carry2
        lax.fori_loop(0, SUB // 16, scale16, 0)

        pltpu.sync_copy(rows_v, acc.at[dst_v.at[j]], add=True)
        return carry

    lax.fori_loop(0, cnt, chunk_body, 0)
    plsc.subcore_barrier()

    pltpu.sync_copy(acc.at[pl.ds(base_n, ROWS_PER_TILE)],
                    out.at[c, pl.ds(base_n, ROWS_PER_TILE)])


def _scatter_gather(xlin, srcs, dsts, ews):
    mesh = plsc.VectorSubcoreMesh(core_axis_name="c", subcore_axis_name="s")
    return pl.kernel(
        _sc_body,
        out_type=jax.ShapeDtypeStruct((NC, N_PAD, D), jnp.float32),
        mesh=mesh,
        scratch_types=[
            pltpu.VMEM_SHARED((N_PAD, D), jnp.float32),
            pltpu.VMEM((SLAB, SUB), jnp.int32),
            pltpu.VMEM((SLAB, SUB), jnp.int32),
            pltpu.VMEM((SLAB, SUB), jnp.float32),
            pltpu.VMEM((SUB, D), jnp.float32),
            pltpu.SemaphoreType.DMA,
        ],
    )(xlin, srcs, dsts, ews)


def _pad_chunks(a2d):
    return jnp.pad(a2d, ((0, CR_PAD - CR), (0, 0)))


def kernel(x, edge_index, edge_weight, W, b):
    src = _pad_chunks(edge_index[0].astype(jnp.int32).reshape(CR, SUB))
    dst = _pad_chunks(edge_index[1].astype(jnp.int32).reshape(CR, SUB))
    ew = _pad_chunks(edge_weight.reshape(CR, SUB))
    x_lin = _matmul(x, W)
    partials = _scatter_gather(x_lin, src, dst, ew)
    return _mix(partials, b)
